# baseline (speedup 1.0000x reference)
import functools

import jax
import jax.numpy as jnp
from jax import lax
from jax.experimental import pallas as pl
from jax.experimental.pallas import tpu as pltpu
from jax.experimental.pallas import tpu_sc as plsc

NC = 2
NS = 16
NW = NC * NS
NBUF = 4
DEPTH = 3
K = 4


def _build(batch, hist, features):
    mesh = plsc.VectorSubcoreMesh(core_axis_name="c", subcore_axis_name="s")
    e_per_w = batch // NW

    @functools.partial(
        pl.kernel,
        mesh=mesh,
        out_type=jax.ShapeDtypeStruct((batch, hist, features), jnp.float32),
        scratch_types=[
            pltpu.VMEM((e_per_w, hist), jnp.int32),
            pltpu.VMEM((NBUF, hist, features), jnp.float32),
            pltpu.SemaphoreType.DMA,
            pltpu.SemaphoreType.DMA,
        ],
    )
    def emb_kernel(table_hbm, idx_hbm, out_hbm, idx_v, rows_v, gsem, wsem):
        wid = lax.axis_index("s") * NC + lax.axis_index("c")
        base = wid * e_per_w
        rows = tuple(rows_v.at[b] for b in range(NBUF))
        pltpu.sync_copy(idx_hbm.at[wid], idx_v)
        for e in range(DEPTH):
            pltpu.async_copy(table_hbm.at[idx_v.at[e]], rows[e], gsem)

        def outer(m, carry):
            for b in range(NBUF):
                e = m * NBUF + b
                pltpu.make_async_copy(
                    table_hbm.at[idx_v.at[e]], rows[b], gsem
                ).wait()
                pltpu.async_copy(rows[b], out_hbm.at[base + e], wsem)

                @pl.when(e + DEPTH < e_per_w)
                def _():
                    @pl.when(e + DEPTH >= NBUF)
                    def _():
                        pltpu.make_async_copy(
                            rows[(b + DEPTH) % NBUF],
                            out_hbm.at[base + e],
                            wsem,
                        ).wait()

                    pltpu.async_copy(
                        table_hbm.at[idx_v.at[e + DEPTH]],
                        rows[(b + DEPTH) % NBUF],
                        gsem,
                    )
            return carry

        lax.fori_loop(0, e_per_w // NBUF, outer, 0)
        for b in range(NBUF):
            pltpu.make_async_copy(rows[b], out_hbm.at[base], wsem).wait()

    return emb_kernel


def kernel(embedding, inputs):
    batch, hist = inputs.shape
    features = embedding.shape[1]
    piece = batch // K
    build = _build(piece, hist, features)
    pieces = []
    for k in range(K):
        idx_k = lax.slice_in_dim(inputs, k * piece, (k + 1) * piece, axis=0)
        idx_k = idx_k.reshape(NW, piece // NW, hist)
        pieces.append(build(embedding, idx_k))
    out = jnp.zeros((batch, hist, features), jnp.float32)
    for k in range(K):
        out = lax.dynamic_update_slice(out, pieces[k], (k * piece, 0, 0))
        out = lax.optimization_barrier(out)
    return out


# speedup vs baseline: 1.7780x; 1.7780x over previous
import functools

import jax
import jax.numpy as jnp
from jax import lax
from jax.experimental import pallas as pl
from jax.experimental.pallas import tpu as pltpu
from jax.experimental.pallas import tpu_sc as plsc

NC = 2
NS = 16
NW = NC * NS
GRP = 2
NBUF = 4
DEPTH = 2


def _build(batch, hist, features):
    mesh = plsc.VectorSubcoreMesh(core_axis_name="c", subcore_axis_name="s")
    e_per_w = batch // NW
    g_per_w = e_per_w // GRP

    @functools.partial(
        pl.kernel,
        mesh=mesh,
        out_type=jax.ShapeDtypeStruct((batch, hist, features), jnp.float32),
        scratch_types=[
            pltpu.VMEM((e_per_w, hist), jnp.int32),
            pltpu.VMEM((NBUF, GRP, hist, features), jnp.float32),
            pltpu.SemaphoreType.DMA,
            pltpu.SemaphoreType.DMA,
        ],
    )
    def emb_kernel(table_hbm, idx_hbm, out_hbm, idx_v, rows_v, gsem, wsem):
        wid = lax.axis_index("s") * NC + lax.axis_index("c")
        base = wid * e_per_w
        rows = tuple(rows_v.at[b] for b in range(NBUF))
        pltpu.sync_copy(idx_hbm.at[wid], idx_v)

        def start_group(g, b):
            for j in range(GRP):
                pltpu.async_copy(
                    table_hbm.at[idx_v.at[g * GRP + j]], rows[b].at[j], gsem
                )

        def wait_group(g, b):
            for j in range(GRP):
                pltpu.make_async_copy(
                    table_hbm.at[idx_v.at[g * GRP + j]], rows[b].at[j], gsem
                ).wait()

        for g in range(DEPTH):
            start_group(g, g)

        def outer(m, carry):
            for b in range(NBUF):
                g = m * NBUF + b
                wait_group(g, b)
                pltpu.async_copy(
                    rows[b], out_hbm.at[pl.ds(base + g * GRP, GRP)], wsem
                )

                @pl.when(g + DEPTH < g_per_w)
                def _():
                    @pl.when(g + DEPTH >= NBUF)
                    def _():
                        pltpu.make_async_copy(
                            rows[(b + DEPTH) % NBUF],
                            out_hbm.at[pl.ds(base + g * GRP, GRP)],
                            wsem,
                        ).wait()

                    start_group(g + DEPTH, (b + DEPTH) % NBUF)
            return carry

        lax.fori_loop(0, g_per_w // NBUF, outer, 0)
        for b in range(NBUF):
            pltpu.make_async_copy(
                rows[b], out_hbm.at[pl.ds(base, GRP)], wsem
            ).wait()

    return emb_kernel


def kernel(embedding, inputs):
    batch, hist = inputs.shape
    features = embedding.shape[1]
    idx = inputs.reshape(NW, batch // NW, hist)
    return _build(batch, hist, features)(embedding, idx)


# speedup vs baseline: 1.7915x; 1.0076x over previous
import functools

import jax
import jax.numpy as jnp
from jax import lax
from jax.experimental import pallas as pl
from jax.experimental.pallas import tpu as pltpu
from jax.experimental.pallas import tpu_sc as plsc

NC = 2
NS = 16
NW = NC * NS
GRP = 4
NBUF = 4
DEPTH = 2


def _build(batch, hist, features):
    mesh = plsc.VectorSubcoreMesh(core_axis_name="c", subcore_axis_name="s")
    e_per_w = batch // NW
    g_per_w = e_per_w // GRP

    @functools.partial(
        pl.kernel,
        mesh=mesh,
        out_type=jax.ShapeDtypeStruct((batch, hist, features), jnp.float32),
        scratch_types=[
            pltpu.VMEM((e_per_w, hist), jnp.int32),
            pltpu.VMEM((NBUF, GRP, hist, features), jnp.float32),
            pltpu.SemaphoreType.DMA,
            pltpu.SemaphoreType.DMA,
        ],
    )
    def emb_kernel(table_hbm, idx_hbm, out_hbm, idx_v, rows_v, gsem, wsem):
        wid = lax.axis_index("s") * NC + lax.axis_index("c")
        base = wid * e_per_w
        rows = tuple(rows_v.at[b] for b in range(NBUF))
        pltpu.sync_copy(idx_hbm.at[wid], idx_v)

        def start_group(g, b):
            for j in range(GRP):
                pltpu.async_copy(
                    table_hbm.at[idx_v.at[g * GRP + j]], rows[b].at[j], gsem
                )

        def wait_group(g, b):
            for j in range(GRP):
                pltpu.make_async_copy(
                    table_hbm.at[idx_v.at[g * GRP + j]], rows[b].at[j], gsem
                ).wait()

        for g in range(DEPTH):
            start_group(g, g)

        def outer(m, carry):
            for b in range(NBUF):
                g = m * NBUF + b
                wait_group(g, b)
                pltpu.async_copy(
                    rows[b], out_hbm.at[pl.ds(base + g * GRP, GRP)], wsem
                )

                @pl.when(g + DEPTH < g_per_w)
                def _():
                    @pl.when(g + DEPTH >= NBUF)
                    def _():
                        pltpu.make_async_copy(
                            rows[(b + DEPTH) % NBUF],
                            out_hbm.at[pl.ds(base + g * GRP, GRP)],
                            wsem,
                        ).wait()

                    start_group(g + DEPTH, (b + DEPTH) % NBUF)
            return carry

        lax.fori_loop(0, g_per_w // NBUF, outer, 0)
        for b in range(NBUF):
            pltpu.make_async_copy(
                rows[b], out_hbm.at[pl.ds(base, GRP)], wsem
            ).wait()

    return emb_kernel


def kernel(embedding, inputs):
    batch, hist = inputs.shape
    features = embedding.shape[1]
    idx = inputs.reshape(NW, batch // NW, hist)
    return _build(batch, hist, features)(embedding, idx)
